# baseline (device time: 25725 ns/iter reference)
import jax
import jax.numpy as jnp
from jax import lax
from jax.experimental import pallas as pl
from jax.experimental.pallas import tpu as pltpu

M = 1024
N = 1024
H = 512
Q = 256
NC = 4
E = Q // NC


def kernel(x):
    def body(x_ref, o_ref, rsa, rsb, xga, xgb, send_sems, recv_sems):
        my_x = lax.axis_index("x")
        my_y = lax.axis_index("y")
        y_nbr = (my_x, 1 - my_y)
        x_nbr = (1 - my_x, my_y)

        a1 = my_y * Q
        b1 = H + my_x * Q
        a_send = (1 - my_y) * Q
        b_send = H + (1 - my_x) * Q

        def ex(src, dst, i, nbr):
            return pltpu.make_async_remote_copy(
                src_ref=src, dst_ref=dst,
                send_sem=send_sems.at[i], recv_sem=recv_sems.at[i],
                device_id=nbr, device_id_type=pl.DeviceIdType.MESH,
            )

        barrier = pltpu.get_barrier_semaphore()
        for nbr in (y_nbr, x_nbr):
            pl.semaphore_signal(
                barrier, inc=1, device_id=nbr,
                device_id_type=pl.DeviceIdType.MESH,
            )

        o_ref[pl.ds(a_send, E), :] = x_ref[0, 0, pl.ds(a_send, E), :].astype(
            jnp.bfloat16)
        a_rs1 = [ex(o_ref.at[pl.ds(a_send + c * E, E)], rsa.at[pl.ds(c * E, E)],
                    c, y_nbr) for c in range(NC)]
        pl.semaphore_wait(barrier, 2)
        a_rs1[0].start()
        o_ref[pl.ds(a_send + E, Q - E), :] = x_ref[
            0, 0, pl.ds(a_send + E, Q - E), :].astype(jnp.bfloat16)
        for c in range(1, NC):
            a_rs1[c].start()
        o_ref[pl.ds(b_send, Q), :] = x_ref[0, 0, pl.ds(b_send, Q), :].astype(
            jnp.bfloat16)
        b_rs1 = [ex(o_ref.at[pl.ds(b_send + c * E, E)], rsb.at[pl.ds(c * E, E)],
                    3 * NC + c, x_nbr) for c in range(NC)]
        for c in range(NC):
            b_rs1[c].start()
        o_ref[pl.ds(a1, Q), :] = x_ref[0, 0, pl.ds(a1, Q), :].astype(
            jnp.bfloat16)
        o_ref[pl.ds(b1, Q), :] = x_ref[0, 0, pl.ds(b1, Q), :].astype(
            jnp.bfloat16)

        a_xg = [ex(o_ref.at[pl.ds(a1 + c * E, E)], xga.at[pl.ds(c * E, E)],
                   NC + c, x_nbr) for c in range(NC)]
        b_xg = [ex(o_ref.at[pl.ds(b1 + c * E, E)], xgb.at[pl.ds(c * E, E)],
                   4 * NC + c, y_nbr) for c in range(NC)]
        for c in range(NC):
            a_rs1[c].wait_recv()
            o_ref[pl.ds(a1 + c * E, E), :] = (
                o_ref[pl.ds(a1 + c * E, E), :] + rsa[pl.ds(c * E, E), :])
            a_xg[c].start()
            b_rs1[c].wait_recv()
            o_ref[pl.ds(b1 + c * E, E), :] = (
                o_ref[pl.ds(b1 + c * E, E), :] + rsb[pl.ds(c * E, E), :])
            b_xg[c].start()

        a_ag = [ex(o_ref.at[pl.ds(a1 + c * E, E)], o_ref.at[pl.ds(a1 + c * E, E)],
                   2 * NC + c, y_nbr) for c in range(NC)]
        b_ag = [ex(o_ref.at[pl.ds(b1 + c * E, E)], o_ref.at[pl.ds(b1 + c * E, E)],
                   5 * NC + c, x_nbr) for c in range(NC)]
        for c in range(NC):
            a_xg[c].wait_recv()
            o_ref[pl.ds(a1 + c * E, E), :] = (
                o_ref[pl.ds(a1 + c * E, E), :] + xga[pl.ds(c * E, E), :])
            a_ag[c].start()
            b_xg[c].wait_recv()
            o_ref[pl.ds(b1 + c * E, E), :] = (
                o_ref[pl.ds(b1 + c * E, E), :] + xgb[pl.ds(c * E, E), :])
            b_ag[c].start()

        for c in range(NC):
            a_ag[c].wait_recv()
            b_ag[c].wait_recv()

        for d in a_rs1 + b_rs1 + a_xg + b_xg + a_ag + b_ag:
            d.wait_send()

    return pl.pallas_call(
        body,
        out_shape=jax.ShapeDtypeStruct((M, N), jnp.bfloat16),
        in_specs=[pl.BlockSpec(memory_space=pltpu.VMEM)],
        out_specs=pl.BlockSpec(memory_space=pltpu.VMEM),
        scratch_shapes=[
            pltpu.VMEM((Q, N), jnp.bfloat16),
            pltpu.VMEM((Q, N), jnp.bfloat16),
            pltpu.VMEM((Q, N), jnp.bfloat16),
            pltpu.VMEM((Q, N), jnp.bfloat16),
            pltpu.SemaphoreType.DMA((6 * NC,)),
            pltpu.SemaphoreType.DMA((6 * NC,)),
        ],
        compiler_params=pltpu.CompilerParams(collective_id=0),
    )(x)


# device time: 25500 ns/iter; 1.0088x vs baseline; 1.0088x over previous
import jax
import jax.numpy as jnp
from jax import lax
from jax.experimental import pallas as pl
from jax.experimental.pallas import tpu as pltpu

M = 1024
N = 1024
H = 512
Q = 256
NC = 2
E = Q // NC


def kernel(x):
    def body(x_ref, o_ref, rsa, rsb, xga, xgb, send_sems, recv_sems):
        my_x = lax.axis_index("x")
        my_y = lax.axis_index("y")
        y_nbr = (my_x, 1 - my_y)
        x_nbr = (1 - my_x, my_y)

        a1 = my_y * Q
        b1 = H + my_x * Q
        a_send = (1 - my_y) * Q
        b_send = H + (1 - my_x) * Q

        def ex(src, dst, i, nbr):
            return pltpu.make_async_remote_copy(
                src_ref=src, dst_ref=dst,
                send_sem=send_sems.at[i], recv_sem=recv_sems.at[i],
                device_id=nbr, device_id_type=pl.DeviceIdType.MESH,
            )

        barrier = pltpu.get_barrier_semaphore()
        for nbr in (y_nbr, x_nbr):
            pl.semaphore_signal(
                barrier, inc=1, device_id=nbr,
                device_id_type=pl.DeviceIdType.MESH,
            )

        o_ref[pl.ds(a_send, E), :] = x_ref[0, 0, pl.ds(a_send, E), :].astype(
            jnp.bfloat16)
        a_rs1 = [ex(o_ref.at[pl.ds(a_send + c * E, E)], rsa.at[pl.ds(c * E, E)],
                    c, y_nbr) for c in range(NC)]
        pl.semaphore_wait(barrier, 2)
        a_rs1[0].start()
        o_ref[pl.ds(a_send + E, Q - E), :] = x_ref[
            0, 0, pl.ds(a_send + E, Q - E), :].astype(jnp.bfloat16)
        for c in range(1, NC):
            a_rs1[c].start()
        o_ref[pl.ds(b_send, Q), :] = x_ref[0, 0, pl.ds(b_send, Q), :].astype(
            jnp.bfloat16)
        b_rs1 = [ex(o_ref.at[pl.ds(b_send + c * E, E)], rsb.at[pl.ds(c * E, E)],
                    3 * NC + c, x_nbr) for c in range(NC)]
        for c in range(NC):
            b_rs1[c].start()
        o_ref[pl.ds(a1, Q), :] = x_ref[0, 0, pl.ds(a1, Q), :].astype(
            jnp.bfloat16)
        o_ref[pl.ds(b1, Q), :] = x_ref[0, 0, pl.ds(b1, Q), :].astype(
            jnp.bfloat16)

        a_xg = [ex(o_ref.at[pl.ds(a1 + c * E, E)], xga.at[pl.ds(c * E, E)],
                   NC + c, x_nbr) for c in range(NC)]
        b_xg = [ex(o_ref.at[pl.ds(b1 + c * E, E)], xgb.at[pl.ds(c * E, E)],
                   4 * NC + c, y_nbr) for c in range(NC)]
        for c in range(NC):
            a_rs1[c].wait_recv()
            o_ref[pl.ds(a1 + c * E, E), :] = (
                o_ref[pl.ds(a1 + c * E, E), :] + rsa[pl.ds(c * E, E), :])
            a_xg[c].start()
            b_rs1[c].wait_recv()
            o_ref[pl.ds(b1 + c * E, E), :] = (
                o_ref[pl.ds(b1 + c * E, E), :] + rsb[pl.ds(c * E, E), :])
            b_xg[c].start()

        a_ag = [ex(o_ref.at[pl.ds(a1 + c * E, E)], o_ref.at[pl.ds(a1 + c * E, E)],
                   2 * NC + c, y_nbr) for c in range(NC)]
        b_ag = [ex(o_ref.at[pl.ds(b1 + c * E, E)], o_ref.at[pl.ds(b1 + c * E, E)],
                   5 * NC + c, x_nbr) for c in range(NC)]
        for c in range(NC):
            a_xg[c].wait_recv()
            o_ref[pl.ds(a1 + c * E, E), :] = (
                o_ref[pl.ds(a1 + c * E, E), :] + xga[pl.ds(c * E, E), :])
            a_ag[c].start()
            b_xg[c].wait_recv()
            o_ref[pl.ds(b1 + c * E, E), :] = (
                o_ref[pl.ds(b1 + c * E, E), :] + xgb[pl.ds(c * E, E), :])
            b_ag[c].start()

        for c in range(NC):
            a_ag[c].wait_recv()
            b_ag[c].wait_recv()

        for d in a_rs1 + b_rs1 + a_xg + b_xg + a_ag + b_ag:
            d.wait_send()

    return pl.pallas_call(
        body,
        out_shape=jax.ShapeDtypeStruct((M, N), jnp.bfloat16),
        in_specs=[pl.BlockSpec(memory_space=pltpu.VMEM)],
        out_specs=pl.BlockSpec(memory_space=pltpu.VMEM),
        scratch_shapes=[
            pltpu.VMEM((Q, N), jnp.bfloat16),
            pltpu.VMEM((Q, N), jnp.bfloat16),
            pltpu.VMEM((Q, N), jnp.bfloat16),
            pltpu.VMEM((Q, N), jnp.bfloat16),
            pltpu.SemaphoreType.DMA((6 * NC,)),
            pltpu.SemaphoreType.DMA((6 * NC,)),
        ],
        compiler_params=pltpu.CompilerParams(collective_id=0),
    )(x)
